# baseline (speedup 1.0000x reference)
import functools

import jax
import jax.numpy as jnp
from jax import lax
from jax.experimental import pallas as pl
from jax.experimental.pallas import tpu as pltpu
from jax.experimental.pallas import tpu_sc as plsc

BATCH = 4096
HIST = 200
EMBED_DIM = 64
TOTAL = BATCH * HIST
NUM_CORES = 2
NUM_SUBCORES = 16
NW = NUM_CORES * NUM_SUBCORES
BPW = TOTAL // NW
IDX_W = 128
K = 4
CH = K * IDX_W
NCHUNK = BPW // CH
NPAIR = NCHUNK // 2

_mesh = plsc.VectorSubcoreMesh(core_axis_name="c", subcore_axis_name="s")


@functools.partial(
    pl.kernel,
    mesh=_mesh,
    out_type=jax.ShapeDtypeStruct((TOTAL, EMBED_DIM), jnp.float32),
    scratch_types=[
        pltpu.VMEM((2, K, IDX_W), jnp.int32),
        pltpu.VMEM((2, CH, EMBED_DIM), jnp.float32),
        pltpu.SemaphoreType.DMA,
        pltpu.SemaphoreType.DMA,
        pltpu.SemaphoreType.DMA,
        pltpu.SemaphoreType.DMA,
    ],
    compiler_params=pltpu.CompilerParams(use_tc_tiling_on_sc=False),
)
def _embed(table_hbm, idx_hbm, out_hbm, idx_v, rows_v, g0, g1, o0, o1):
    wid = lax.axis_index("s") * NUM_CORES + lax.axis_index("c")
    base_row = wid * (BPW // IDX_W)
    base_out = wid * BPW
    gsem = (g0, g1)
    osem = (o0, o1)

    def fire_chunk(c, b):
        pltpu.sync_copy(idx_hbm.at[pl.ds(base_row + c * K, K)], idx_v.at[b])
        return [
            pltpu.async_copy(
                table_hbm.at[idx_v.at[b].at[j]],
                rows_v.at[b].at[pl.ds(j * IDX_W, IDX_W)],
                gsem[b],
            )
            for j in range(K)
        ]

    def drain_out(b):
        pltpu.make_async_copy(
            out_hbm.at[pl.ds(0, CH)], rows_v.at[b], osem[b]
        ).wait()

    def body(g, carry):
        c0 = 2 * g

        @pl.when(g > 0)
        def _():
            drain_out(0)
            drain_out(1)

        gathers0 = fire_chunk(c0, 0)
        gathers1 = fire_chunk(c0 + 1, 1)
        for cp in gathers0:
            cp.wait()
        pltpu.async_copy(
            rows_v.at[0], out_hbm.at[pl.ds(base_out + c0 * CH, CH)], osem[0]
        )
        for cp in gathers1:
            cp.wait()
        pltpu.async_copy(
            rows_v.at[1], out_hbm.at[pl.ds(base_out + (c0 + 1) * CH, CH)], osem[1]
        )
        return carry

    lax.fori_loop(0, NPAIR, body, 0)
    drain_out(0)
    drain_out(1)


def kernel(vectors, table):
    idx = vectors.reshape(TOTAL // IDX_W, IDX_W)
    out = _embed(table, idx)
    return out.reshape(BATCH, HIST, EMBED_DIM)


# speedup vs baseline: 1.0111x; 1.0111x over previous
import functools

import jax
import jax.numpy as jnp
from jax import lax
from jax.experimental import pallas as pl
from jax.experimental.pallas import tpu as pltpu
from jax.experimental.pallas import tpu_sc as plsc

BATCH = 4096
HIST = 200
EMBED_DIM = 64
TOTAL = BATCH * HIST
NUM_CORES = 2
NUM_SUBCORES = 16
NW = NUM_CORES * NUM_SUBCORES
BPW = TOTAL // NW
IDX_W = 128
IDX_ROWS = BPW // IDX_W
K = 5
CH = K * IDX_W
NCHUNK = BPW // CH

_mesh = plsc.VectorSubcoreMesh(core_axis_name="c", subcore_axis_name="s")


@functools.partial(
    pl.kernel,
    mesh=_mesh,
    out_type=jax.ShapeDtypeStruct((TOTAL, EMBED_DIM), jnp.float32),
    scratch_types=[
        pltpu.VMEM((IDX_ROWS, IDX_W), jnp.int32),
        pltpu.VMEM((2, CH, EMBED_DIM), jnp.float32),
        pltpu.SemaphoreType.DMA,
        pltpu.SemaphoreType.DMA,
        pltpu.SemaphoreType.DMA,
        pltpu.SemaphoreType.DMA,
    ],
    compiler_params=pltpu.CompilerParams(use_tc_tiling_on_sc=False),
)
def _embed(table_hbm, idx_hbm, out_hbm, idx_v, rows_v, g0, g1, o0, o1):
    wid = lax.axis_index("s") * NUM_CORES + lax.axis_index("c")
    base_row = wid * IDX_ROWS
    base_out = wid * BPW
    gsem = (g0, g1)
    osem = (o0, o1)

    def fire_g(c, b):
        for j in range(K):
            pltpu.async_copy(
                table_hbm.at[idx_v.at[c * K + j]],
                rows_v.at[b].at[pl.ds(j * IDX_W, IDX_W)],
                gsem[b],
            )

    def wait_g(b):
        for _ in range(K):
            pltpu.make_async_copy(
                out_hbm.at[pl.ds(0, IDX_W)],
                rows_v.at[b].at[pl.ds(0, IDX_W)],
                gsem[b],
            ).wait()

    def fire_o(c, b):
        pltpu.async_copy(
            rows_v.at[b], out_hbm.at[pl.ds(base_out + c * CH, CH)], osem[b]
        )

    def drain_o(b):
        pltpu.make_async_copy(
            out_hbm.at[pl.ds(0, CH)], rows_v.at[b], osem[b]
        ).wait()

    pltpu.sync_copy(idx_hbm.at[pl.ds(base_row, IDX_ROWS)], idx_v)

    def step(c, b):
        wait_g(1 - b)
        fire_o(c - 1, 1 - b)
        drain_o(b)
        fire_g(c, b)

    fire_g(0, 0)
    wait_g(0)
    fire_o(0, 0)
    fire_g(1, 1)

    def body(g, carry):
        step(2 * g, 0)
        step(2 * g + 1, 1)
        return carry

    lax.fori_loop(1, NCHUNK // 2, body, 0)

    wait_g(1)
    fire_o(NCHUNK - 1, 1)
    drain_o(0)
    drain_o(1)


def kernel(vectors, table):
    idx = vectors.reshape(TOTAL // IDX_W, IDX_W)
    out = _embed(table, idx)
    return out.reshape(BATCH, HIST, EMBED_DIM)
